# baseline (speedup 1.0000x reference)
import functools

import jax
import jax.numpy as jnp
from jax import lax
from jax.experimental import pallas as pl
from jax.experimental.pallas import tpu as pltpu
from jax.experimental.pallas import tpu_sc as plsc


@functools.cache
def _pos_embed_copy(num_rows, dim, dtype):
    info = plsc.get_sparse_core_info()
    nw = info.num_cores * info.num_subcores
    assert num_rows % nw == 0, (num_rows, nw)
    rows_per_w = num_rows // nw
    mesh = plsc.VectorSubcoreMesh(core_axis_name="c", subcore_axis_name="s")

    nchunks = 4
    assert rows_per_w % nchunks == 0
    ch = rows_per_w // nchunks

    @functools.partial(
        pl.kernel,
        mesh=mesh,
        out_type=jax.ShapeDtypeStruct((num_rows, dim), dtype),
        scratch_types=(
            [pltpu.VMEM((ch, dim), dtype) for _ in range(nchunks)]
            + [pltpu.SemaphoreType.DMA for _ in range(2 * nchunks)]
        ),
    )
    def k(table_hbm, out_hbm, *scratch):
        bufs = scratch[:nchunks]
        in_sems = scratch[nchunks : 2 * nchunks]
        out_sems = scratch[2 * nchunks :]
        wid = lax.axis_index("s") * info.num_cores + lax.axis_index("c")
        base = wid * rows_per_w
        gets = [
            pltpu.async_copy(
                table_hbm.at[pl.ds(base + i * ch, ch)], bufs[i], in_sems[i]
            )
            for i in range(nchunks)
        ]
        puts = []
        for i in range(nchunks):
            gets[i].wait()
            puts.append(
                pltpu.async_copy(
                    bufs[i], out_hbm.at[pl.ds(base + i * ch, ch)], out_sems[i]
                )
            )
        for p in puts:
            p.wait()

    return k


def kernel(seq_embeds, table):
    len_seq = seq_embeds.shape[-2]
    pos_embeds = _pos_embed_copy(len_seq, table.shape[-1], table.dtype)(table)
    if seq_embeds.ndim == 3:
        pos_embeds = pos_embeds[None]
    return pos_embeds
